# baseline (speedup 1.0000x reference)
import functools
import jax
import jax.numpy as jnp
import numpy as np
from jax.experimental import pallas as pl
from jax.experimental.pallas import tpu as pltpu
from jax.experimental.pallas import tpu_sc as plsc

_N_NODES = 10000
_E_EDGES = 160000
_D_NODE = 128
_D_EDGE = 16
_IN_CH = _D_NODE + _D_EDGE
_HID = 128
_OUT = 128
_EPS = 1e-5
_PCAP = 24 * _E_EDGES

_TILE_E = 640

_NW = 32
_SLAB = 24576
_NSTARTS = 10112


def _sget(ref, i):
    return ref[pl.ds(i, 16)][0]


def _pair_softmax_weights(a_s_pad, a_d_pad, starts_pad, meta):
    mesh = plsc.VectorSubcoreMesh(core_axis_name="c", subcore_axis_name="s")

    @functools.partial(
        pl.kernel, mesh=mesh,
        out_type=jax.ShapeDtypeStruct((_NW, _SLAB), jnp.float32),
        compiler_params=pltpu.CompilerParams(needs_layout_passes=False),
        scratch_types=[
            pltpu.VMEM((16,), jnp.int32),
            pltpu.VMEM((_NSTARTS,), jnp.int32),
            pltpu.VMEM((_SLAB + 128,), jnp.float32),
            pltpu.VMEM((_SLAB + 128,), jnp.float32),
            pltpu.VMEM((_SLAB,), jnp.float32),
        ],
    )
    def k(as_hbm, ad_hbm, starts_hbm, meta_hbm, w_hbm,
          meta_v, starts_v, as_v, ad_v, w_v):
        wid = jax.lax.axis_index("s") * 2 + jax.lax.axis_index("c")
        pltpu.sync_copy(meta_hbm.at[wid], meta_v)
        mv = meta_v[...]
        slab0 = pl.multiple_of(mv[0], 8)
        g0 = mv[3]
        g1 = mv[4]
        pltpu.sync_copy(starts_hbm, starts_v)
        pltpu.sync_copy(as_hbm.at[pl.ds(slab0, _SLAB)], as_v.at[pl.ds(0, _SLAB)])
        pltpu.sync_copy(ad_hbm.at[pl.ds(slab0, _SLAB)], ad_v.at[pl.ds(0, _SLAB)])

        lane = jax.lax.iota(jnp.int32, 16)
        zeros16 = jnp.zeros((16,), jnp.float32)

        def zero_body(kk, _):
            w_v[pl.ds(kk * 16, 16)] = zeros16
            return 0
        jax.lax.fori_loop(0, _SLAB // 16, zero_body, 0)

        def group_body(g, _):
            s = _sget(starts_v, g)
            s_next = _sget(starts_v, g + 1)
            c = s_next - s
            b = s - slab0
            hi = jnp.minimum(b + c - 1, _SLAB - 1)
            n_i = (c + 15) // 16

            def gather_as(i):
                idx = b + i * 16 + lane
                msk = idx <= hi
                v = plsc.load_gather(as_v, [jnp.minimum(idx, hi)])
                return idx, msk, v

            def amax_body(i, acc):
                _, msk, v = gather_as(i)
                return jnp.maximum(acc, jnp.max(jnp.where(msk, v, -3e38)))
            a_max = jax.lax.fori_loop(0, n_i, amax_body, jnp.float32(-3e38))

            def j_body(jj, _):
                bj = jnp.minimum(b + jj, _SLAB - 1)
                a_dj = _sget(ad_v, bj)
                x_m = a_max + a_dj
                m_j = jnp.where(x_m > 0, x_m, 0.2 * x_m)

                def den_body(i, acc):
                    _, msk, v = gather_as(i)
                    x = v + a_dj
                    t = jnp.exp(jnp.where(x > 0, x, 0.2 * x) - m_j)
                    return acc + jnp.sum(jnp.where(msk, t, 0.0), axis=0)
                den = jax.lax.fori_loop(0, n_i, den_body, jnp.float32(0.0))
                den_v = jnp.full((16,), den, jnp.float32)

                def w_body(i, _):
                    idx, msk, v = gather_as(i)
                    x = v + a_dj
                    t = jnp.exp(jnp.where(x > 0, x, 0.2 * x) - m_j) / den_v
                    idxc = jnp.minimum(idx, hi)
                    wv = plsc.load_gather(w_v, [idxc])
                    plsc.store_scatter(w_v, [idxc], wv + t, mask=msk)
                    return 0
                jax.lax.fori_loop(0, n_i, w_body, 0)
                return 0
            jax.lax.fori_loop(0, c, j_body, 0)
            return 0

        jax.lax.fori_loop(g0, g1, group_body, 0)
        pltpu.sync_copy(w_v, w_hbm.at[wid])

    return k(a_s_pad, a_d_pad, starts_pad, meta)


def _stats_kernel(h0_ref, m2_ref, mu_ref):
    i = pl.program_id(0)
    h0 = h0_ref[...]
    m2 = jnp.dot(h0.T, h0, preferred_element_type=jnp.float32)
    mu = jnp.broadcast_to(jnp.sum(h0, axis=0, keepdims=True), (8, _IN_CH))

    @pl.when(i == 0)
    def _():
        m2_ref[...] = m2
        mu_ref[...] = mu

    @pl.when(i != 0)
    def _():
        m2_ref[...] += m2
        mu_ref[...] += mu


def _stats_pass(h0):
    n_tiles = _E_EDGES // _TILE_E
    return pl.pallas_call(
        _stats_kernel,
        grid=(n_tiles,),
        in_specs=[pl.BlockSpec((_TILE_E, _IN_CH), lambda i: (i, 0))],
        out_specs=[
            pl.BlockSpec((_IN_CH, _IN_CH), lambda i: (0, 0)),
            pl.BlockSpec((8, _IN_CH), lambda i: (0, 0)),
        ],
        out_shape=[
            jax.ShapeDtypeStruct((_IN_CH, _IN_CH), jnp.float32),
            jax.ShapeDtypeStruct((8, _IN_CH), jnp.float32),
        ],
    )(h0)


def _dense_edge_kernel(h0_ref, w1_ref, beff_ref, s1_ref, b1_ref,
                       vs_ref, vd_ref, h1_ref, as_ref, ad_ref):
    h0 = h0_ref[...]
    h = jnp.dot(h0, w1_ref[...], preferred_element_type=jnp.float32)
    h = h + beff_ref[...]
    h1 = jnp.maximum(h * s1_ref[...] + b1_ref[...], 0.0)
    h1_ref[...] = h1
    as_ref[...] = jnp.dot(h1, vs_ref[...], preferred_element_type=jnp.float32)
    ad_ref[...] = jnp.dot(h1, vd_ref[...], preferred_element_type=jnp.float32)


def _dense_edge_pass(h0, W1eff, beff, s1, b1p, v_s, v_d):
    n_tiles = _E_EDGES // _TILE_E
    return pl.pallas_call(
        _dense_edge_kernel,
        grid=(n_tiles,),
        in_specs=[
            pl.BlockSpec((_TILE_E, _IN_CH), lambda i: (i, 0)),
            pl.BlockSpec((_IN_CH, _HID), lambda i: (0, 0)),
            pl.BlockSpec((1, _HID), lambda i: (0, 0)),
            pl.BlockSpec((1, _HID), lambda i: (0, 0)),
            pl.BlockSpec((1, _HID), lambda i: (0, 0)),
            pl.BlockSpec((_HID, 128), lambda i: (0, 0)),
            pl.BlockSpec((_HID, 128), lambda i: (0, 0)),
        ],
        out_specs=[
            pl.BlockSpec((_TILE_E, _HID), lambda i: (i, 0)),
            pl.BlockSpec((_TILE_E, 128), lambda i: (i, 0)),
            pl.BlockSpec((_TILE_E, 128), lambda i: (i, 0)),
        ],
        out_shape=[
            jax.ShapeDtypeStruct((_E_EDGES, _HID), jnp.float32),
            jax.ShapeDtypeStruct((_E_EDGES, 128), jnp.float32),
            jax.ShapeDtypeStruct((_E_EDGES, 128), jnp.float32),
        ],
    )(h0, W1eff, beff, s1, b1p, v_s, v_d)


def kernel(node_attr, edge_attr, gamma0, beta0, W1, gamma1, beta1, Wg,
           att_src, att_dst, bias_g, gamma2, beta2, edge_index):
    idx_r = edge_index[0].astype(jnp.int32)
    E = _E_EDGES
    N = _N_NODES

    g_sorted, order = jax.lax.sort(
        (idx_r, jnp.arange(E, dtype=jnp.int32)), num_keys=1)
    starts = jnp.searchsorted(g_sorted, jnp.arange(N + 1, dtype=jnp.int32),
                              side='left').astype(jnp.int32)
    counts = starts[1:] - starts[:-1]
    cf = counts.astype(jnp.float32)

    h0 = jnp.concatenate([jnp.take(node_attr, g_sorted, axis=0),
                          jnp.take(edge_attr, order, axis=0)], axis=1)
    M2s, mus = _stats_pass(h0)
    mu0 = mus[0] / E
    M2 = M2s / E
    var0 = jnp.diagonal(M2) - mu0 * mu0
    s0 = gamma0 / jnp.sqrt(var0 + _EPS)
    W1eff = W1 * s0[:, None]
    beff = (beta0 - mu0 * s0) @ W1
    mu1 = mu0 @ W1eff + beff
    ex2_1 = jnp.einsum('ij,ik,kj->j', W1eff, M2, W1eff) \
        + 2.0 * beff * (mu0 @ W1eff) + beff * beff
    var1 = ex2_1 - mu1 * mu1
    s1 = gamma1 / jnp.sqrt(var1 + _EPS)
    b1p = beta1 - mu1 * s1

    v_s = Wg @ att_src
    v_d = Wg @ att_dst

    h1, a_s2, a_d2 = _dense_edge_pass(
        h0, W1eff, beff[None, :], s1[None, :], b1p[None, :],
        jnp.tile(v_s[:, None], (1, 128)), jnp.tile(v_d[:, None], (1, 128)))
    a_s = a_s2[:, 0]
    a_d = a_d2[:, 0]

    c2 = counts.astype(jnp.int64) * counts.astype(jnp.int64)
    cum = jnp.cumsum(c2)
    total2 = cum[-1]
    tw = (jnp.arange(1, _NW, dtype=jnp.int64) * total2) // _NW
    gb = jnp.concatenate([
        jnp.zeros((1,), jnp.int32),
        jnp.searchsorted(cum, tw, side='left').astype(jnp.int32) + 1,
        jnp.full((1,), N, jnp.int32)])
    gb = jnp.minimum(gb, N)
    e_bounds = starts[gb]
    slab0 = (e_bounds[:-1] // 8) * 8
    meta = jnp.zeros((_NW, 16), jnp.int32)
    meta = meta.at[:, 0].set(slab0)
    meta = meta.at[:, 1].set(e_bounds[:-1])
    meta = meta.at[:, 2].set(e_bounds[1:])
    meta = meta.at[:, 3].set(gb[:-1])
    meta = meta.at[:, 4].set(gb[1:])

    a_s_pad = jnp.concatenate([a_s, jnp.zeros((_SLAB,), jnp.float32)])
    a_d_pad = jnp.concatenate([a_d, jnp.zeros((_SLAB,), jnp.float32)])
    starts_pad = jnp.concatenate(
        [starts, jnp.full((_NSTARTS - N - 1,), E, jnp.int32)])

    w_slabs = _pair_softmax_weights(a_s_pad, a_d_pad, starts_pad, meta)

    t_all = jnp.arange(E, dtype=jnp.int32)
    w_of_t = jnp.clip(jnp.searchsorted(e_bounds, t_all, side='right') - 1,
                      0, _NW - 1).astype(jnp.int32)
    flat_idx = w_of_t * _SLAB + (t_all - slab0[w_of_t])
    w_sorted = w_slabs.reshape(-1)[flat_idx]

    Z = jax.ops.segment_sum(w_sorted[:, None] * h1, g_sorted,
                            num_segments=N)
    h3 = Z @ Wg + cf[:, None] * bias_g[None, :]
    mu3 = jnp.mean(h3, axis=0)
    var3 = jnp.mean((h3 - mu3) ** 2, axis=0)
    return (h3 - mu3) / jnp.sqrt(var3 + _EPS) * gamma2 + beta2


# speedup vs baseline: 3.1910x; 3.1910x over previous
import functools
import jax
import jax.numpy as jnp
import numpy as np
from jax.experimental import pallas as pl
from jax.experimental.pallas import tpu as pltpu
from jax.experimental.pallas import tpu_sc as plsc

_N_NODES = 10000
_E_EDGES = 160000
_D_NODE = 128
_D_EDGE = 16
_IN_CH = _D_NODE + _D_EDGE
_HID = 128
_OUT = 128
_EPS = 1e-5
_PCAP = 24 * _E_EDGES

_TILE_E = 640

_NW = 32
_SLAB = 24576
_NSTARTS = 10112


def _sget(ref, i):
    return ref[pl.ds(i, 16)][0]


def _pair_softmax_weights(a_s_pad, a_d_pad, starts_pad, meta):
    mesh = plsc.VectorSubcoreMesh(core_axis_name="c", subcore_axis_name="s")

    @functools.partial(
        pl.kernel, mesh=mesh,
        out_type=jax.ShapeDtypeStruct((_NW, _SLAB), jnp.float32),
        compiler_params=pltpu.CompilerParams(needs_layout_passes=False),
        scratch_types=[
            pltpu.VMEM((16,), jnp.int32),
            pltpu.VMEM((_NSTARTS,), jnp.int32),
            pltpu.VMEM((_SLAB + 128,), jnp.float32),
            pltpu.VMEM((_SLAB + 128,), jnp.float32),
            pltpu.VMEM((_SLAB,), jnp.float32),
        ],
    )
    def k(as_hbm, ad_hbm, starts_hbm, meta_hbm, w_hbm,
          meta_v, starts_v, as_v, ad_v, w_v):
        wid = jax.lax.axis_index("s") * 2 + jax.lax.axis_index("c")
        pltpu.sync_copy(meta_hbm.at[wid], meta_v)
        mv = meta_v[...]
        slab0 = pl.multiple_of(mv[0], 8)
        g0 = mv[3]
        g1 = mv[4]
        pltpu.sync_copy(starts_hbm, starts_v)
        pltpu.sync_copy(as_hbm.at[pl.ds(slab0, _SLAB)], as_v.at[pl.ds(0, _SLAB)])
        pltpu.sync_copy(ad_hbm.at[pl.ds(slab0, _SLAB)], ad_v.at[pl.ds(0, _SLAB)])

        lane = jax.lax.iota(jnp.int32, 16)
        zeros16 = jnp.zeros((16,), jnp.float32)

        def zero_body(kk, _):
            w_v[pl.ds(kk * 16, 16)] = zeros16
            return 0
        jax.lax.fori_loop(0, _SLAB // 16, zero_body, 0)

        def group_body(g, _):
            s = _sget(starts_v, g)
            s_next = _sget(starts_v, g + 1)
            c = s_next - s
            b = s - slab0
            hi = jnp.minimum(b + c - 1, _SLAB - 1)
            n_i = (c + 15) // 16

            def gather_as(i):
                idx = b + i * 16 + lane
                msk = idx <= hi
                v = plsc.load_gather(as_v, [jnp.minimum(idx, hi)])
                return idx, msk, v

            def amax_body(i, acc):
                _, msk, v = gather_as(i)
                return jnp.maximum(acc, jnp.max(jnp.where(msk, v, -3e38)))
            a_max = jax.lax.fori_loop(0, n_i, amax_body, jnp.float32(-3e38))

            def j_body(jj, _):
                bj = jnp.minimum(b + jj, _SLAB - 1)
                a_dj = _sget(ad_v, bj)
                x_m = a_max + a_dj
                m_j = jnp.where(x_m > 0, x_m, 0.2 * x_m)

                def den_body(i, acc):
                    _, msk, v = gather_as(i)
                    x = v + a_dj
                    t = jnp.exp(jnp.where(x > 0, x, 0.2 * x) - m_j)
                    return acc + jnp.sum(jnp.where(msk, t, 0.0), axis=0)
                den = jax.lax.fori_loop(0, n_i, den_body, jnp.float32(0.0))
                den_v = jnp.full((16,), den, jnp.float32)

                def w_body(i, _):
                    idx, msk, v = gather_as(i)
                    x = v + a_dj
                    t = jnp.exp(jnp.where(x > 0, x, 0.2 * x) - m_j) / den_v
                    idxc = jnp.minimum(idx, hi)
                    wv = plsc.load_gather(w_v, [idxc])
                    plsc.store_scatter(w_v, [idxc], wv + t, mask=msk)
                    return 0
                jax.lax.fori_loop(0, n_i, w_body, 0)
                return 0
            jax.lax.fori_loop(0, c, j_body, 0)
            return 0

        jax.lax.fori_loop(g0, g1, group_body, 0)
        pltpu.sync_copy(w_v, w_hbm.at[wid])

    return k(a_s_pad, a_d_pad, starts_pad, meta)


def _stats_kernel(h0_ref, m2_ref, mu_ref):
    i = pl.program_id(0)
    h0 = h0_ref[...]
    m2 = jnp.dot(h0.T, h0, preferred_element_type=jnp.float32)
    mu = jnp.broadcast_to(jnp.sum(h0, axis=0, keepdims=True), (8, _IN_CH))

    @pl.when(i == 0)
    def _():
        m2_ref[...] = m2
        mu_ref[...] = mu

    @pl.when(i != 0)
    def _():
        m2_ref[...] += m2
        mu_ref[...] += mu


def _stats_pass(h0):
    n_tiles = _E_EDGES // _TILE_E
    return pl.pallas_call(
        _stats_kernel,
        grid=(n_tiles,),
        in_specs=[pl.BlockSpec((_TILE_E, _IN_CH), lambda i: (i, 0))],
        out_specs=[
            pl.BlockSpec((_IN_CH, _IN_CH), lambda i: (0, 0)),
            pl.BlockSpec((8, _IN_CH), lambda i: (0, 0)),
        ],
        out_shape=[
            jax.ShapeDtypeStruct((_IN_CH, _IN_CH), jnp.float32),
            jax.ShapeDtypeStruct((8, _IN_CH), jnp.float32),
        ],
    )(h0)


def _dense_edge_kernel(h0_ref, w1_ref, beff_ref, s1_ref, b1_ref,
                       vs_ref, vd_ref, h1_ref, as_ref, ad_ref):
    h0 = h0_ref[...]
    h = jnp.dot(h0, w1_ref[...], preferred_element_type=jnp.float32)
    h = h + beff_ref[...]
    h1 = jnp.maximum(h * s1_ref[...] + b1_ref[...], 0.0)
    h1_ref[...] = h1
    as_ref[...] = jnp.dot(h1, vs_ref[...], preferred_element_type=jnp.float32)
    ad_ref[...] = jnp.dot(h1, vd_ref[...], preferred_element_type=jnp.float32)


def _dense_edge_pass(h0, W1eff, beff, s1, b1p, v_s, v_d):
    n_tiles = _E_EDGES // _TILE_E
    return pl.pallas_call(
        _dense_edge_kernel,
        grid=(n_tiles,),
        in_specs=[
            pl.BlockSpec((_TILE_E, _IN_CH), lambda i: (i, 0)),
            pl.BlockSpec((_IN_CH, _HID), lambda i: (0, 0)),
            pl.BlockSpec((1, _HID), lambda i: (0, 0)),
            pl.BlockSpec((1, _HID), lambda i: (0, 0)),
            pl.BlockSpec((1, _HID), lambda i: (0, 0)),
            pl.BlockSpec((_HID, 128), lambda i: (0, 0)),
            pl.BlockSpec((_HID, 128), lambda i: (0, 0)),
        ],
        out_specs=[
            pl.BlockSpec((_TILE_E, _HID), lambda i: (i, 0)),
            pl.BlockSpec((_TILE_E, 128), lambda i: (i, 0)),
            pl.BlockSpec((_TILE_E, 128), lambda i: (i, 0)),
        ],
        out_shape=[
            jax.ShapeDtypeStruct((_E_EDGES, _HID), jnp.float32),
            jax.ShapeDtypeStruct((_E_EDGES, 128), jnp.float32),
            jax.ShapeDtypeStruct((_E_EDGES, 128), jnp.float32),
        ],
    )(h0, W1eff, beff, s1, b1p, v_s, v_d)


def kernel(node_attr, edge_attr, gamma0, beta0, W1, gamma1, beta1, Wg,
           att_src, att_dst, bias_g, gamma2, beta2, edge_index):
    idx_r = edge_index[0].astype(jnp.int32)
    E = _E_EDGES
    N = _N_NODES

    g_sorted, order = jax.lax.sort(
        (idx_r, jnp.arange(E, dtype=jnp.int32)), num_keys=1)
    cf = jax.ops.segment_sum(jnp.ones((E,), jnp.float32), idx_r,
                             num_segments=N)
    counts = cf.astype(jnp.int32)
    starts = jnp.concatenate([jnp.zeros((1,), jnp.int32),
                              jnp.cumsum(counts)]).astype(jnp.int32)

    h0 = jnp.concatenate([jnp.take(node_attr, g_sorted, axis=0),
                          jnp.take(edge_attr, order, axis=0)], axis=1)
    M2s, mus = _stats_pass(h0)
    mu0 = mus[0] / E
    M2 = M2s / E
    var0 = jnp.diagonal(M2) - mu0 * mu0
    s0 = gamma0 / jnp.sqrt(var0 + _EPS)
    W1eff = W1 * s0[:, None]
    beff = (beta0 - mu0 * s0) @ W1
    mu1 = mu0 @ W1eff + beff
    ex2_1 = jnp.einsum('ij,ik,kj->j', W1eff, M2, W1eff) \
        + 2.0 * beff * (mu0 @ W1eff) + beff * beff
    var1 = ex2_1 - mu1 * mu1
    s1 = gamma1 / jnp.sqrt(var1 + _EPS)
    b1p = beta1 - mu1 * s1

    v_s = Wg @ att_src
    v_d = Wg @ att_dst

    h1, a_s2, a_d2 = _dense_edge_pass(
        h0, W1eff, beff[None, :], s1[None, :], b1p[None, :],
        jnp.tile(v_s[:, None], (1, 128)), jnp.tile(v_d[:, None], (1, 128)))
    a_s = a_s2[:, 0]
    a_d = a_d2[:, 0]

    c2 = counts.astype(jnp.int64) * counts.astype(jnp.int64)
    cum = jnp.cumsum(c2)
    total2 = cum[-1]
    tw = (jnp.arange(1, _NW, dtype=jnp.int64) * total2) // _NW
    gb = jnp.concatenate([
        jnp.zeros((1,), jnp.int32),
        jnp.searchsorted(cum, tw, side='left').astype(jnp.int32) + 1,
        jnp.full((1,), N, jnp.int32)])
    gb = jnp.minimum(gb, N)
    e_bounds = starts[gb]
    slab0 = (e_bounds[:-1] // 8) * 8
    meta = jnp.zeros((_NW, 16), jnp.int32)
    meta = meta.at[:, 0].set(slab0)
    meta = meta.at[:, 1].set(e_bounds[:-1])
    meta = meta.at[:, 2].set(e_bounds[1:])
    meta = meta.at[:, 3].set(gb[:-1])
    meta = meta.at[:, 4].set(gb[1:])

    a_s_pad = jnp.concatenate([a_s, jnp.zeros((_SLAB,), jnp.float32)])
    a_d_pad = jnp.concatenate([a_d, jnp.zeros((_SLAB,), jnp.float32)])
    starts_pad = jnp.concatenate(
        [starts, jnp.full((_NSTARTS - N - 1,), E, jnp.int32)])

    w_slabs = _pair_softmax_weights(a_s_pad, a_d_pad, starts_pad, meta)

    t_all = jnp.arange(E, dtype=jnp.int32)
    inb = (t_all[:, None] >= e_bounds[None, :-1]) \
        & (t_all[:, None] < e_bounds[None, 1:])
    w_of_t = jnp.sum(inb * jnp.arange(_NW, dtype=jnp.int32)[None, :], axis=1)
    slab0_of_t = jnp.sum(inb * slab0[None, :], axis=1)
    flat_idx = w_of_t * _SLAB + (t_all - slab0_of_t)
    w_sorted = w_slabs.reshape(-1)[flat_idx]

    Z = jax.ops.segment_sum(w_sorted[:, None] * h1, g_sorted,
                            num_segments=N)
    h3 = Z @ Wg + cf[:, None] * bias_g[None, :]
    mu3 = jnp.mean(h3, axis=0)
    var3 = jnp.mean((h3 - mu3) ** 2, axis=0)
    return (h3 - mu3) / jnp.sqrt(var3 + _EPS) * gamma2 + beta2
